# baseline (speedup 1.0000x reference)
import functools

import jax
import jax.numpy as jnp
from jax import lax
from jax.experimental import pallas as pl
from jax.experimental.pallas import tpu as pltpu
from jax.experimental.pallas import tpu_sc as plsc

BATCH = 16384
N_SPACE = 8
LANES = 16

_info = plsc.get_sparse_core_info()
_NC, _NS = _info.num_cores, _info.num_subcores
_NW = _NC * _NS
_B_PER_W = BATCH // _NW
_VREGS = _B_PER_W // LANES

_mesh = plsc.VectorSubcoreMesh(core_axis_name="c", subcore_axis_name="s")


@functools.partial(
    pl.kernel,
    mesh=_mesh,
    out_type=jax.ShapeDtypeStruct((BATCH,), jnp.float32),
    compiler_params=pltpu.CompilerParams(
        needs_layout_passes=False,
        disable_bounds_checks=True,
        disable_semaphore_checks=True,
        skip_device_barrier=True,
    ),
    scratch_types=[
        pltpu.VMEM((_B_PER_W,), jnp.int32),
        pltpu.VMEM((_B_PER_W,), jnp.int32),
        pltpu.VMEM((_B_PER_W,), jnp.int32),
        pltpu.VMEM((_B_PER_W,), jnp.int32),
        pltpu.VMEM((N_SPACE, N_SPACE), jnp.float32),
        pltpu.VMEM((LANES,), jnp.float32),
        pltpu.VMEM((_B_PER_W,), jnp.float32),
        pltpu.SemaphoreType.DMA,
        pltpu.SemaphoreType.DMA,
        pltpu.SemaphoreType.DMA,
        pltpu.SemaphoreType.DMA,
        pltpu.SemaphoreType.DMA,
        pltpu.SemaphoreType.DMA,
    ],
)
def _sc_kernel(xt_hbm, xpt_hbm, alpha_hbm, beta_hbm, out_hbm,
               x0v, x1v, xp0v, xp1v, av, bv, ov,
               sem0, sem1, sem2, sem3, sem4, sem5):
    wid = lax.axis_index("s") * _NC + lax.axis_index("c")
    base = wid * _B_PER_W

    c0 = pltpu.async_copy(xt_hbm.at[0, pl.ds(base, _B_PER_W)], x0v, sem0)
    c1 = pltpu.async_copy(xt_hbm.at[1, pl.ds(base, _B_PER_W)], x1v, sem1)
    c2 = pltpu.async_copy(xpt_hbm.at[0, pl.ds(base, _B_PER_W)], xp0v, sem2)
    c3 = pltpu.async_copy(xpt_hbm.at[1, pl.ds(base, _B_PER_W)], xp1v, sem3)
    c4 = pltpu.async_copy(alpha_hbm, av, sem4)
    c5 = pltpu.async_copy(beta_hbm, bv.at[pl.ds(0, 1)], sem5)
    c0.wait()
    c1.wait()
    c2.wait()
    c3.wait()
    c4.wait()
    c5.wait()

    beta = bv[...][0]

    @pl.loop(0, _VREGS, unroll=4)
    def _compute(j):
        sl = pl.ds(j * LANES, LANES)
        x0 = x0v[sl]
        x1 = x1v[sl]
        xp0 = xp0v[sl]
        xp1 = xp1v[sl]
        al = plsc.load_gather(av, [xp1, x1])
        dt = jnp.abs(x0 - xp0).astype(jnp.float32)
        ov[sl] = al * beta * jnp.exp(-beta * dt)

    pltpu.sync_copy(ov, out_hbm.at[pl.ds(base, _B_PER_W)])


def kernel(x, xp, alpha, beta):
    return _sc_kernel(x.T, xp.T, alpha, beta)
